# baseline (speedup 1.0000x reference)
import functools

import jax
import jax.numpy as jnp
from jax import lax
from jax.experimental import pallas as pl
from jax.experimental.pallas import tpu as pltpu
from jax.experimental.pallas import tpu_sc as plsc

NUM_CORES = 2
NUM_SUBCORES = 16
NUM_WORKERS = NUM_CORES * NUM_SUBCORES
LANES = 16
BATCH = 16384
DIM = 64
WIDTH = 128
BPW = BATCH // NUM_WORKERS
CHUNK = 128
NCHUNK = BPW // CHUNK
GROUPS = CHUNK // LANES

_MESH = plsc.VectorSubcoreMesh(
    core_axis_name="c", subcore_axis_name="s",
    num_cores=NUM_CORES, num_subcores=NUM_SUBCORES,
)


@functools.partial(
    pl.kernel,
    out_type=jax.ShapeDtypeStruct((BATCH,), jnp.float32),
    mesh=_MESH,
    scratch_types=[
        pltpu.VMEM((CHUNK,), jnp.int32),
        pltpu.VMEM((CHUNK,), jnp.int32),
        pltpu.VMEM((CHUNK,), jnp.int32),
        pltpu.VMEM((CHUNK,), jnp.int32),
        pltpu.VMEM((CHUNK,), jnp.int32),
        pltpu.VMEM((CHUNK,), jnp.int32),
        pltpu.VMEM((CHUNK, WIDTH), jnp.float32),
        pltpu.VMEM((CHUNK, WIDTH), jnp.float32),
        pltpu.VMEM((CHUNK, WIDTH), jnp.float32),
        pltpu.VMEM((CHUNK, WIDTH), jnp.float32),
        pltpu.VMEM((CHUNK, WIDTH), jnp.float32),
        pltpu.VMEM((CHUNK, WIDTH), jnp.float32),
        pltpu.VMEM((BPW,), jnp.float32),
        pltpu.SemaphoreType.DMA,
    ],
    compiler_params=pltpu.CompilerParams(needs_layout_passes=False),
)
def _complex_score_sc(h_hbm, r_hbm, t_hbm, ent_re2, ent_im2, rel_re2, rel_im2,
                      out_hbm, hidx_v, ridx_v, tidx_v,
                      hrow_v, rrow_v, trow_v,
                      hre_v, him_v, rre_v, rim_v, tre_v, tim_v,
                      out_v, sem):
    wid = lax.axis_index("s") * NUM_CORES + lax.axis_index("c")
    base = wid * BPW

    for c in range(NCHUNK):
        off = base + c * CHUNK
        pltpu.sync_copy(h_hbm.at[pl.ds(off, CHUNK)], hidx_v)
        pltpu.sync_copy(r_hbm.at[pl.ds(off, CHUNK)], ridx_v)
        pltpu.sync_copy(t_hbm.at[pl.ds(off, CHUNK)], tidx_v)

        def row_body(g, _):
            sl = pl.ds(g * LANES, LANES)
            hrow_v[sl] = hidx_v[sl] >> 1
            rrow_v[sl] = ridx_v[sl] >> 1
            trow_v[sl] = tidx_v[sl] >> 1
            return 0

        lax.fori_loop(0, GROUPS, row_body, 0)

        pltpu.async_copy(ent_re2.at[hrow_v], hre_v, sem)
        pltpu.async_copy(ent_im2.at[hrow_v], him_v, sem)
        pltpu.async_copy(rel_re2.at[rrow_v], rre_v, sem)
        pltpu.async_copy(rel_im2.at[rrow_v], rim_v, sem)
        pltpu.async_copy(ent_re2.at[trow_v], tre_v, sem)
        pltpu.async_copy(ent_im2.at[trow_v], tim_v, sem)
        for buf in (hre_v, him_v, rre_v, rim_v, tre_v, tim_v):
            pltpu.make_async_copy(
                ent_re2.at[pl.ds(0, CHUNK), :], buf, sem).wait()

        def group_body(g, _, c=c):
            sl = pl.ds(g * LANES, LANES)
            rows = g * LANES + lax.iota(jnp.int32, LANES)
            hoff = (hidx_v[sl] & 1) * DIM
            roff = (ridx_v[sl] & 1) * DIM
            toff = (tidx_v[sl] & 1) * DIM

            def dim_body(d, acc):
                hre = plsc.load_gather(hre_v, [rows, hoff + d])
                him = plsc.load_gather(him_v, [rows, hoff + d])
                rre = plsc.load_gather(rre_v, [rows, roff + d])
                rim = plsc.load_gather(rim_v, [rows, roff + d])
                tre = plsc.load_gather(tre_v, [rows, toff + d])
                tim = plsc.load_gather(tim_v, [rows, toff + d])
                re_hr = hre * rre - him * rim
                im_hr = hre * rim + him * rre
                return acc + tre * re_hr + tim * im_hr

            acc = lax.fori_loop(0, DIM, dim_body, jnp.zeros((LANES,), jnp.float32))
            out_v[pl.ds(c * CHUNK + g * LANES, LANES)] = acc
            return 0

        lax.fori_loop(0, GROUPS, group_body, 0)

    pltpu.sync_copy(out_v, out_hbm.at[pl.ds(base, BPW)])


def kernel(triples, ent_re, ent_im, rel_re, rel_im):
    h = triples[:, 0].astype(jnp.int32)
    r = triples[:, 1].astype(jnp.int32)
    t = triples[:, 2].astype(jnp.int32)
    ent_re2 = ent_re.reshape(-1, WIDTH)
    ent_im2 = ent_im.reshape(-1, WIDTH)
    rel_re2 = rel_re.reshape(-1, WIDTH)
    rel_im2 = rel_im.reshape(-1, WIDTH)
    return _complex_score_sc(h, r, t, ent_re2, ent_im2, rel_re2, rel_im2)
